# Initial kernel scaffold; baseline (speedup 1.0000x reference)
import jax
import jax.numpy as jnp
from jax.experimental import pallas as pl


def kernel(xyz_fine, xyz_coarse, feats_fine, feats_coarse, W1, gamma1, beta1, W2, gamma2, beta2):
    raise NotImplementedError("write your pallas kernel here")


# speedup vs baseline: 8.0021x; 8.0021x over previous
import functools

import jax
import jax.numpy as jnp
from jax import lax
from jax.experimental import pallas as pl
from jax.experimental.pallas import tpu as pltpu
from jax.experimental.pallas import tpu_sc as plsc

N = 16384
M = 4096
CF = 128
CC = 256
K = 3
BLK = 256
GRID = N // BLK


def _knn_body(q_ref, s_ref, idx_ref, w_ref):
    q = q_ref[...]
    s = s_ref[...]
    qq = (q[:, 0:1] * q[:, 0:1] + q[:, 1:2] * q[:, 1:2]) + q[:, 2:3] * q[:, 2:3]
    ss = (s[0:1] * s[0:1] + s[1:2] * s[1:2]) + s[2:3] * s[2:3]
    qb = q.astype(jnp.bfloat16).astype(jnp.float32)
    sb = s.astype(jnp.bfloat16).astype(jnp.float32)
    qs = (qb[:, 0:1] * sb[0:1] + qb[:, 1:2] * sb[1:2]) + qb[:, 2:3] * sb[2:3]
    d2r = (qq + ss) - 2.0 * qs
    d2e = jnp.zeros((BLK, M), jnp.float32)
    for c in range(3):
        diff = q[:, c : c + 1] - s[c : c + 1, :]
        d2e = d2e + diff * diff
    col = lax.broadcasted_iota(jnp.int32, (BLK, M), 1)
    inf = jnp.float32(float("inf"))
    vals = []
    idxs = []
    for _ in range(K):
        mj = jnp.min(d2r, axis=1, keepdims=True)
        aj = jnp.min(jnp.where(d2r == mj, col, M), axis=1, keepdims=True)
        ej = jnp.min(jnp.where(col == aj, d2e, inf), axis=1, keepdims=True)
        vals.append(ej)
        idxs.append(aj)
        d2r = jnp.where(col == aj, inf, d2r)
    v = jnp.concatenate(vals, axis=1)
    a = jnp.concatenate(idxs, axis=1)
    d = jnp.maximum(jnp.sqrt(v), 1e-8)
    w = 1.0 / d
    w = w / jnp.sum(w, axis=1, keepdims=True)
    idx_ref[...] = a
    w_ref[...] = w


def _knn(q, s):
    return pl.pallas_call(
        _knn_body,
        grid=(GRID,),
        in_specs=[
            pl.BlockSpec((BLK, 3), lambda i: (i, 0)),
            pl.BlockSpec((3, M), lambda i: (0, 0)),
        ],
        out_specs=[
            pl.BlockSpec((BLK, K), lambda i: (i, 0)),
            pl.BlockSpec((BLK, K), lambda i: (i, 0)),
        ],
        out_shape=[
            jax.ShapeDtypeStruct((N, K), jnp.int32),
            jax.ShapeDtypeStruct((N, K), jnp.float32),
        ],
    )(q, s)


_NC = 2
_NS = 16
_NW = _NC * _NS
_ROWS = K * N
_RPW = _ROWS // _NW
_CHUNK = 128
_NCHUNK = _RPW // _CHUNK


def _sc_gather(table, idx_flat):
    mesh = plsc.VectorSubcoreMesh(core_axis_name="c", subcore_axis_name="s")

    @functools.partial(
        pl.kernel,
        mesh=mesh,
        out_type=jax.ShapeDtypeStruct((_ROWS, CC), jnp.float32),
        scratch_types=[
            pltpu.VMEM((_RPW,), jnp.int32),
            pltpu.VMEM((_CHUNK, CC), jnp.float32),
            pltpu.VMEM((_CHUNK, CC), jnp.float32),
            pltpu.SemaphoreType.DMA,
            pltpu.SemaphoreType.DMA,
        ],
    )
    def k(table_hbm, idx_hbm, out_hbm, idx_v, buf0, buf1, sem0, sem1):
        wid = lax.axis_index("s") * _NC + lax.axis_index("c")
        base = wid * _RPW
        pltpu.sync_copy(idx_hbm.at[pl.ds(base, _RPW)], idx_v)
        bufs = (buf0, buf1)
        sems = (sem0, sem1)
        cps = [None, None]
        cps[0] = pltpu.async_copy(
            table_hbm.at[idx_v.at[pl.ds(0, _CHUNK)]], bufs[0], sems[0]
        )
        for t in range(_NCHUNK):
            nxt = (t + 1) % 2
            if t + 1 < _NCHUNK:
                cps[nxt] = pltpu.async_copy(
                    table_hbm.at[idx_v.at[pl.ds((t + 1) * _CHUNK, _CHUNK)]],
                    bufs[nxt],
                    sems[nxt],
                )
            cps[t % 2].wait()
            pltpu.sync_copy(
                bufs[t % 2], out_hbm.at[pl.ds(base + t * _CHUNK, _CHUNK)]
            )

    return k(table, idx_flat)


def _mlp1_body(g_ref, w_ref, ff_ref, w1a_ref, w1b_ref, y1_ref, s_ref, q_ref):
    i = pl.program_id(0)
    interp = (
        w_ref[:, 0:1] * g_ref[0]
        + w_ref[:, 1:2] * g_ref[1]
        + w_ref[:, 2:3] * g_ref[2]
    )
    y = jnp.dot(interp, w1a_ref[...], preferred_element_type=jnp.float32)
    y = y + jnp.dot(ff_ref[...], w1b_ref[...], preferred_element_type=jnp.float32)
    y1_ref[...] = y

    @pl.when(i == 0)
    def _init():
        s_ref[...] = jnp.zeros_like(s_ref)
        q_ref[...] = jnp.zeros_like(q_ref)

    s_ref[...] += jnp.sum(y, axis=0, keepdims=True)
    q_ref[...] += jnp.sum(y * y, axis=0, keepdims=True)


def _mlp1(g, w, ff_t, w1a_t, w1b_t):
    return pl.pallas_call(
        _mlp1_body,
        grid=(GRID,),
        in_specs=[
            pl.BlockSpec((K, BLK, CC), lambda i: (0, i, 0)),
            pl.BlockSpec((BLK, K), lambda i: (i, 0)),
            pl.BlockSpec((BLK, CF), lambda i: (i, 0)),
            pl.BlockSpec((CC, 256), lambda i: (0, 0)),
            pl.BlockSpec((CF, 256), lambda i: (0, 0)),
        ],
        out_specs=[
            pl.BlockSpec((BLK, 256), lambda i: (i, 0)),
            pl.BlockSpec((1, 256), lambda i: (0, 0)),
            pl.BlockSpec((1, 256), lambda i: (0, 0)),
        ],
        out_shape=[
            jax.ShapeDtypeStruct((N, 256), jnp.float32),
            jax.ShapeDtypeStruct((1, 256), jnp.float32),
            jax.ShapeDtypeStruct((1, 256), jnp.float32),
        ],
    )(g, w, ff_t, w1a_t, w1b_t)


def _mlp2_body(y1_ref, a_ref, b_ref, w2_ref, y2_ref, s_ref, q_ref):
    i = pl.program_id(0)
    h = jnp.maximum(y1_ref[...] * a_ref[...] + b_ref[...], 0.0)
    y = jnp.dot(h, w2_ref[...], preferred_element_type=jnp.float32)
    y2_ref[...] = y

    @pl.when(i == 0)
    def _init():
        s_ref[...] = jnp.zeros_like(s_ref)
        q_ref[...] = jnp.zeros_like(q_ref)

    s_ref[...] += jnp.sum(y, axis=0, keepdims=True)
    q_ref[...] += jnp.sum(y * y, axis=0, keepdims=True)


def _mlp2(y1, a1, b1, w2_t):
    return pl.pallas_call(
        _mlp2_body,
        grid=(GRID,),
        in_specs=[
            pl.BlockSpec((BLK, 256), lambda i: (i, 0)),
            pl.BlockSpec((1, 256), lambda i: (0, 0)),
            pl.BlockSpec((1, 256), lambda i: (0, 0)),
            pl.BlockSpec((256, 256), lambda i: (0, 0)),
        ],
        out_specs=[
            pl.BlockSpec((BLK, 256), lambda i: (i, 0)),
            pl.BlockSpec((1, 256), lambda i: (0, 0)),
            pl.BlockSpec((1, 256), lambda i: (0, 0)),
        ],
        out_shape=[
            jax.ShapeDtypeStruct((N, 256), jnp.float32),
            jax.ShapeDtypeStruct((1, 256), jnp.float32),
            jax.ShapeDtypeStruct((1, 256), jnp.float32),
        ],
    )(y1, a1, b1, w2_t)


def _norm_body(y2_ref, a_ref, b_ref, out_ref):
    o = jnp.maximum(y2_ref[...] * a_ref[...] + b_ref[...], 0.0)
    out_ref[...] = o.T


def _norm(y2, a2, b2):
    return pl.pallas_call(
        _norm_body,
        grid=(GRID,),
        in_specs=[
            pl.BlockSpec((BLK, 256), lambda i: (i, 0)),
            pl.BlockSpec((1, 256), lambda i: (0, 0)),
            pl.BlockSpec((1, 256), lambda i: (0, 0)),
        ],
        out_specs=pl.BlockSpec((256, BLK), lambda i: (0, i)),
        out_shape=jax.ShapeDtypeStruct((256, N), jnp.float32),
    )(y2, a2, b2)


def _bn_coefs(s, q, gamma, beta):
    mean = s / N
    var = q / N - mean * mean
    a = gamma[None, :] / jnp.sqrt(var + 1e-5)
    b = beta[None, :] - mean * a
    return a, b


def kernel(xyz_fine, xyz_coarse, feats_fine, feats_coarse,
           W1, gamma1, beta1, W2, gamma2, beta2):
    q = xyz_fine[0].T
    s = xyz_coarse[0]
    idx, w = _knn(q, s)

    table = feats_coarse[0].T
    idx_flat = idx.T.reshape(_ROWS)
    g_rows = _sc_gather(table, idx_flat)
    g = g_rows.reshape(K, N, CC)

    ff_t = feats_fine[0].T
    w1a_t = W1[:, :CC].T
    w1b_t = W1[:, CC:].T
    y1, s1, q1 = _mlp1(g, w, ff_t, w1a_t, w1b_t)
    a1, b1 = _bn_coefs(s1, q1, gamma1, beta1)

    y2, s2, q2 = _mlp2(y1, a1, b1, W2.T)
    a2, b2 = _bn_coefs(s2, q2, gamma2, beta2)

    out = _norm(y2, a2, b2)
    return out[None]


# speedup vs baseline: 10.6823x; 1.3349x over previous
import functools

import jax
import jax.numpy as jnp
from jax import lax
from jax.experimental import pallas as pl
from jax.experimental.pallas import tpu as pltpu
from jax.experimental.pallas import tpu_sc as plsc

N = 16384
M = 4096
CF = 128
CC = 256
K = 3
CD = 128
BLK = 256
GRID = N // BLK
CW = 128
NCH = M // CW


def _knn_body(q_ref, s_ref, idx_ref):
    q = q_ref[...]
    s = s_ref[...]
    qq = (q[:, 0:1] * q[:, 0:1] + q[:, 1:2] * q[:, 1:2]) + q[:, 2:3] * q[:, 2:3]
    ss = (s[0:1] * s[0:1] + s[1:2] * s[1:2]) + s[2:3] * s[2:3]
    qb = q.astype(jnp.bfloat16)
    sb = s.astype(jnp.bfloat16)
    qs = jnp.dot(qb, sb, preferred_element_type=jnp.float32)
    inf = jnp.float32(float("inf"))
    big = jnp.int32(2**30)
    v1 = jnp.full((BLK, CW), inf, jnp.float32)
    v2 = v1
    v3 = v1
    i1 = jnp.full((BLK, CW), big, jnp.int32)
    i2 = i1
    i3 = i1
    lane = lax.broadcasted_iota(jnp.int32, (BLK, CW), 1)
    for c in range(NCH):
        sl = slice(c * CW, (c + 1) * CW)
        v = (qq + ss[:, sl]) - 2.0 * qs[:, sl]
        iv = lane + (c * CW)
        lt1 = v < v1
        lt2 = v < v2
        lt3 = v < v3
        v3n = jnp.where(lt3, jnp.where(lt2, v2, v), v3)
        i3n = jnp.where(lt3, jnp.where(lt2, i2, iv), i3)
        v2n = jnp.where(lt2, jnp.where(lt1, v1, v), v2)
        i2n = jnp.where(lt2, jnp.where(lt1, i1, iv), i2)
        v1 = jnp.where(lt1, v, v1)
        i1 = jnp.where(lt1, iv, i1)
        v2, v3, i2, i3 = v2n, v3n, i2n, i3n
    V = jnp.concatenate([v1, v2, v3], axis=1)
    I = jnp.concatenate([i1, i2, i3], axis=1)
    idxs = []
    for _ in range(K):
        m = jnp.min(V, axis=1, keepdims=True)
        a = jnp.min(jnp.where(V == m, I, big), axis=1, keepdims=True)
        idxs.append(a)
        V = jnp.where((V == m) & (I == a), inf, V)
    idx_ref[...] = jnp.concatenate(idxs, axis=1)


def _knn(q, s):
    return pl.pallas_call(
        _knn_body,
        grid=(GRID,),
        in_specs=[
            pl.BlockSpec((BLK, 3), lambda i: (i, 0)),
            pl.BlockSpec((3, M), lambda i: (0, 0)),
        ],
        out_specs=pl.BlockSpec((BLK, K), lambda i: (i, 0)),
        out_shape=jax.ShapeDtypeStruct((N, K), jnp.int32),
    )(q, s)


_NC = 2
_NS = 16
_NW = _NC * _NS
_ROWS = K * N
_RPW = _ROWS // _NW
_CHUNK = 128
_NCHUNK = _RPW // _CHUNK


def _sc_gather(table, ctab, idx_flat):
    mesh = plsc.VectorSubcoreMesh(core_axis_name="c", subcore_axis_name="s")

    @functools.partial(
        pl.kernel,
        mesh=mesh,
        out_type=[
            jax.ShapeDtypeStruct((_ROWS, CC), jnp.float32),
            jax.ShapeDtypeStruct((_ROWS, CD), jnp.float32),
        ],
        scratch_types=[
            pltpu.VMEM((_RPW,), jnp.int32),
            pltpu.VMEM((_CHUNK, CC), jnp.float32),
            pltpu.VMEM((_CHUNK, CC), jnp.float32),
            pltpu.VMEM((_CHUNK, CD), jnp.float32),
            pltpu.VMEM((_CHUNK, CD), jnp.float32),
            pltpu.SemaphoreType.DMA,
            pltpu.SemaphoreType.DMA,
            pltpu.SemaphoreType.DMA,
            pltpu.SemaphoreType.DMA,
        ],
    )
    def k(table_hbm, ctab_hbm, idx_hbm, outf_hbm, outc_hbm,
          idx_v, fb0, fb1, cb0, cb1, fs0, fs1, cs0, cs1):
        wid = lax.axis_index("s") * _NC + lax.axis_index("c")
        base = wid * _RPW
        pltpu.sync_copy(idx_hbm.at[pl.ds(base, _RPW)], idx_v)
        fbufs = (fb0, fb1)
        cbufs = (cb0, cb1)
        fsems = (fs0, fs1)
        csems = (cs0, cs1)

        def issue(t, b):
            isl = idx_v.at[pl.ds(t * _CHUNK, _CHUNK)]
            fcp = pltpu.async_copy(table_hbm.at[isl], fbufs[b], fsems[b])
            ccp = pltpu.async_copy(ctab_hbm.at[isl], cbufs[b], csems[b])
            return fcp, ccp

        cps = [None, None]
        cps[0] = issue(0, 0)
        for t in range(_NCHUNK):
            b = t % 2
            nb = (t + 1) % 2
            if t + 1 < _NCHUNK:
                cps[nb] = issue(t + 1, nb)
            cps[b][0].wait()
            cps[b][1].wait()
            osl = pl.ds(base + t * _CHUNK, _CHUNK)
            pltpu.sync_copy(fbufs[b], outf_hbm.at[osl])
            pltpu.sync_copy(cbufs[b], outc_hbm.at[osl])

    return k(table, ctab, idx_flat)


def _mlp1_body(g_ref, c_ref, qp_ref, ff_ref, w1a_ref, w1b_ref,
               y1_ref, s_ref, q_ref):
    i = pl.program_id(0)
    qp = qp_ref[...]
    ws = []
    for j in range(K):
        cj = c_ref[j]
        t0 = qp[:, 0:1] - cj[:, 0:1]
        t1 = qp[:, 1:2] - cj[:, 1:2]
        t2 = qp[:, 2:3] - cj[:, 2:3]
        d2 = (t0 * t0 + t1 * t1) + t2 * t2
        d = jnp.maximum(jnp.sqrt(d2), 1e-8)
        ws.append(1.0 / d)
    tot = (ws[0] + ws[1]) + ws[2]
    wn = [wj / tot for wj in ws]
    interp = (wn[0] * g_ref[0] + wn[1] * g_ref[1]) + wn[2] * g_ref[2]
    y = jnp.dot(interp, w1a_ref[...], preferred_element_type=jnp.float32)
    y = y + jnp.dot(ff_ref[...], w1b_ref[...], preferred_element_type=jnp.float32)
    y1_ref[...] = y

    @pl.when(i == 0)
    def _init():
        s_ref[...] = jnp.zeros_like(s_ref)
        q_ref[...] = jnp.zeros_like(q_ref)

    s_ref[...] += jnp.sum(y, axis=0, keepdims=True)
    q_ref[...] += jnp.sum(y * y, axis=0, keepdims=True)


def _mlp1(g, c, q, ff_t, w1a_t, w1b_t):
    return pl.pallas_call(
        _mlp1_body,
        grid=(GRID,),
        in_specs=[
            pl.BlockSpec((K, BLK, CC), lambda i: (0, i, 0)),
            pl.BlockSpec((K, BLK, CD), lambda i: (0, i, 0)),
            pl.BlockSpec((BLK, 3), lambda i: (i, 0)),
            pl.BlockSpec((BLK, CF), lambda i: (i, 0)),
            pl.BlockSpec((CC, 256), lambda i: (0, 0)),
            pl.BlockSpec((CF, 256), lambda i: (0, 0)),
        ],
        out_specs=[
            pl.BlockSpec((BLK, 256), lambda i: (i, 0)),
            pl.BlockSpec((1, 256), lambda i: (0, 0)),
            pl.BlockSpec((1, 256), lambda i: (0, 0)),
        ],
        out_shape=[
            jax.ShapeDtypeStruct((N, 256), jnp.float32),
            jax.ShapeDtypeStruct((1, 256), jnp.float32),
            jax.ShapeDtypeStruct((1, 256), jnp.float32),
        ],
    )(g, c, q, ff_t, w1a_t, w1b_t)


def _mlp2_body(y1_ref, a_ref, b_ref, w2_ref, y2_ref, s_ref, q_ref):
    i = pl.program_id(0)
    h = jnp.maximum(y1_ref[...] * a_ref[...] + b_ref[...], 0.0)
    y = jnp.dot(h, w2_ref[...], preferred_element_type=jnp.float32)
    y2_ref[...] = y

    @pl.when(i == 0)
    def _init():
        s_ref[...] = jnp.zeros_like(s_ref)
        q_ref[...] = jnp.zeros_like(q_ref)

    s_ref[...] += jnp.sum(y, axis=0, keepdims=True)
    q_ref[...] += jnp.sum(y * y, axis=0, keepdims=True)


def _mlp2(y1, a1, b1, w2_t):
    return pl.pallas_call(
        _mlp2_body,
        grid=(GRID,),
        in_specs=[
            pl.BlockSpec((BLK, 256), lambda i: (i, 0)),
            pl.BlockSpec((1, 256), lambda i: (0, 0)),
            pl.BlockSpec((1, 256), lambda i: (0, 0)),
            pl.BlockSpec((256, 256), lambda i: (0, 0)),
        ],
        out_specs=[
            pl.BlockSpec((BLK, 256), lambda i: (i, 0)),
            pl.BlockSpec((1, 256), lambda i: (0, 0)),
            pl.BlockSpec((1, 256), lambda i: (0, 0)),
        ],
        out_shape=[
            jax.ShapeDtypeStruct((N, 256), jnp.float32),
            jax.ShapeDtypeStruct((1, 256), jnp.float32),
            jax.ShapeDtypeStruct((1, 256), jnp.float32),
        ],
    )(y1, a1, b1, w2_t)


def _norm_body(y2_ref, a_ref, b_ref, out_ref):
    o = jnp.maximum(y2_ref[...] * a_ref[...] + b_ref[...], 0.0)
    out_ref[...] = o.T


def _norm(y2, a2, b2):
    return pl.pallas_call(
        _norm_body,
        grid=(GRID,),
        in_specs=[
            pl.BlockSpec((BLK, 256), lambda i: (i, 0)),
            pl.BlockSpec((1, 256), lambda i: (0, 0)),
            pl.BlockSpec((1, 256), lambda i: (0, 0)),
        ],
        out_specs=pl.BlockSpec((256, BLK), lambda i: (0, i)),
        out_shape=jax.ShapeDtypeStruct((256, N), jnp.float32),
    )(y2, a2, b2)


def _bn_coefs(s, q, gamma, beta):
    mean = s / N
    var = q / N - mean * mean
    a = gamma[None, :] / jnp.sqrt(var + 1e-5)
    b = beta[None, :] - mean * a
    return a, b


def kernel(xyz_fine, xyz_coarse, feats_fine, feats_coarse,
           W1, gamma1, beta1, W2, gamma2, beta2):
    q = xyz_fine[0].T
    s = xyz_coarse[0]
    idx = _knn(q, s)

    table = feats_coarse[0].T
    ctab = jnp.pad(s.T, ((0, 0), (0, CD - 3)))
    idx_flat = idx.T.reshape(_ROWS)
    g_rows, c_rows = _sc_gather(table, ctab, idx_flat)
    g = g_rows.reshape(K, N, CC)
    c = c_rows.reshape(K, N, CD)

    ff_t = feats_fine[0].T
    w1a_t = W1[:, :CC].T
    w1b_t = W1[:, CC:].T
    y1, s1, q1 = _mlp1(g, c, q, ff_t, w1a_t, w1b_t)
    a1, b1 = _bn_coefs(s1, q1, gamma1, beta1)

    y2, s2, q2 = _mlp2(y1, a1, b1, W2.T)
    a2, b2 = _bn_coefs(s2, q2, gamma2, beta2)

    out = _norm(y2, a2, b2)
    return out[None]
